# Initial kernel scaffold; baseline (speedup 1.0000x reference)
import jax
import jax.numpy as jnp
from jax.experimental import pallas as pl


def kernel(x, edge_index, W_enc, b_enc, W_dec, b_dec):
    raise NotImplementedError("write your pallas kernel here")


# speedup vs baseline: 16.6474x; 16.6474x over previous
import functools

import jax
import jax.numpy as jnp
from jax import lax
from jax.experimental import pallas as pl
from jax.experimental.pallas import tpu as pltpu
from jax.experimental.pallas import tpu_sc as plsc

N = 10000
E = 320000
D = 128

NC = 2
NS = 16
NW = NC * NS
EPW = E // NW
CH = 128
NFULL = EPW // CH
TAIL = EPW - NFULL * CH
ZR = 640
ZL = N - 15 * ZR
DPAD = 10240
DSTRIPE = DPAD // NS

BLK = 1000
GRID = N // BLK


def _sc_mesh():
    return plsc.VectorSubcoreMesh(core_axis_name="c", subcore_axis_name="s")


@functools.partial(
    pl.kernel,
    mesh=_sc_mesh(),
    out_type=[jax.ShapeDtypeStruct((DPAD,), jnp.float32),
              jax.ShapeDtypeStruct((DPAD,), jnp.float32)],
    scratch_types=[
        pltpu.VMEM((CH,), jnp.int32),
        pltpu.VMEM((TAIL,), jnp.int32),
        pltpu.VMEM((CH,), jnp.float32),
        pltpu.VMEM((TAIL,), jnp.float32),
        pltpu.VMEM_SHARED((DPAD,), jnp.float32),
    ],
)
def _deg_kernel(dst_hbm, ones_hbm, zeros_hbm, out0_hbm, out1_hbm,
                idx_v, idxt_v, ones_v, onest_v, deg_sh):
    c = lax.axis_index("c")
    s = lax.axis_index("s")
    base = (s * NC + c) * EPW
    pltpu.sync_copy(ones_hbm.at[pl.ds(0, CH)], ones_v)
    pltpu.sync_copy(ones_hbm.at[pl.ds(0, TAIL)], onest_v)
    pltpu.sync_copy(zeros_hbm, deg_sh.at[pl.ds(s * DSTRIPE, DSTRIPE)])
    plsc.subcore_barrier()

    def body(k, carry):
        pltpu.sync_copy(dst_hbm.at[pl.ds(base + k * CH, CH)], idx_v)
        pltpu.sync_copy(ones_v, deg_sh.at[idx_v], add=True)
        return carry

    lax.fori_loop(0, NFULL, body, 0)
    pltpu.sync_copy(dst_hbm.at[pl.ds(base + NFULL * CH, TAIL)], idxt_v)
    pltpu.sync_copy(onest_v, deg_sh.at[idxt_v], add=True)
    plsc.subcore_barrier()

    @pl.when(c == 0)
    def _():
        pltpu.sync_copy(deg_sh.at[pl.ds(s * DSTRIPE, DSTRIPE)],
                        out0_hbm.at[pl.ds(s * DSTRIPE, DSTRIPE)])

    @pl.when(c == 1)
    def _():
        pltpu.sync_copy(deg_sh.at[pl.ds(s * DSTRIPE, DSTRIPE)],
                        out1_hbm.at[pl.ds(s * DSTRIPE, DSTRIPE)])


@functools.partial(
    pl.kernel,
    mesh=_sc_mesh(),
    out_type=[jax.ShapeDtypeStruct((N, D), jnp.float32),
              jax.ShapeDtypeStruct((N, D), jnp.float32)],
    scratch_types=[
        pltpu.VMEM((CH,), jnp.int32),
        pltpu.VMEM((CH,), jnp.int32),
        pltpu.VMEM((TAIL,), jnp.int32),
        pltpu.VMEM((TAIL,), jnp.int32),
        pltpu.VMEM((CH, D), jnp.float32),
        pltpu.VMEM((TAIL, D), jnp.float32),
        pltpu.VMEM_SHARED((N, D), jnp.float32),
        pltpu.SemaphoreType.DMA,
    ],
)
def _spmm_kernel(g_hbm, src_hbm, dst_hbm, zeros_hbm, out0_hbm, out1_hbm,
                 src_v, dst_v, srct_v, dstt_v, rows_v, rowst_v, acc_sh, sem):
    c = lax.axis_index("c")
    s = lax.axis_index("s")
    base = (s * NC + c) * EPW

    @pl.when(s < 15)
    def _():
        pltpu.sync_copy(zeros_hbm.at[pl.ds(0, ZR)], acc_sh.at[pl.ds(s * ZR, ZR)])

    @pl.when(s == 15)
    def _():
        pltpu.sync_copy(zeros_hbm.at[pl.ds(0, ZL)], acc_sh.at[pl.ds(15 * ZR, ZL)])

    plsc.subcore_barrier()

    def body(k, carry):
        off = base + k * CH
        pltpu.sync_copy(src_hbm.at[pl.ds(off, CH)], src_v)
        pltpu.sync_copy(dst_hbm.at[pl.ds(off, CH)], dst_v)
        pltpu.async_copy(g_hbm.at[src_v], rows_v, sem).wait()
        pltpu.sync_copy(rows_v, acc_sh.at[dst_v], add=True)
        return carry

    lax.fori_loop(0, NFULL, body, 0)
    offt = base + NFULL * CH
    pltpu.sync_copy(src_hbm.at[pl.ds(offt, TAIL)], srct_v)
    pltpu.sync_copy(dst_hbm.at[pl.ds(offt, TAIL)], dstt_v)
    pltpu.async_copy(g_hbm.at[srct_v], rowst_v, sem).wait()
    pltpu.sync_copy(rowst_v, acc_sh.at[dstt_v], add=True)
    plsc.subcore_barrier()

    @pl.when(jnp.logical_and(c == 0, s < 15))
    def _():
        pltpu.sync_copy(acc_sh.at[pl.ds(s * ZR, ZR)], out0_hbm.at[pl.ds(s * ZR, ZR)])

    @pl.when(jnp.logical_and(c == 0, s == 15))
    def _():
        pltpu.sync_copy(acc_sh.at[pl.ds(15 * ZR, ZL)], out0_hbm.at[pl.ds(15 * ZR, ZL)])

    @pl.when(jnp.logical_and(c == 1, s < 15))
    def _():
        pltpu.sync_copy(acc_sh.at[pl.ds(s * ZR, ZR)], out1_hbm.at[pl.ds(s * ZR, ZR)])

    @pl.when(jnp.logical_and(c == 1, s == 15))
    def _():
        pltpu.sync_copy(acc_sh.at[pl.ds(15 * ZR, ZL)], out1_hbm.at[pl.ds(15 * ZR, ZL)])


def _dinv_block(d0_ref, d1_ref):
    return lax.rsqrt(d0_ref[...] + d1_ref[...] + 1.0)


def _mm_scale_body(d0_ref, d1_ref, x_ref, w_ref, out_ref):
    dinv = _dinv_block(d0_ref, d1_ref)
    out_ref[...] = jnp.dot(x_ref[...], w_ref[...],
                           preferred_element_type=jnp.float32) * dinv


_mm_scale = pl.pallas_call(
    _mm_scale_body,
    grid=(GRID,),
    in_specs=[
        pl.BlockSpec((BLK, 1), lambda i: (i, 0)),
        pl.BlockSpec((BLK, 1), lambda i: (i, 0)),
        pl.BlockSpec((BLK, D), lambda i: (i, 0)),
        pl.BlockSpec((D, D), lambda i: (0, 0)),
    ],
    out_specs=pl.BlockSpec((BLK, D), lambda i: (i, 0)),
    out_shape=jax.ShapeDtypeStruct((N, D), jnp.float32),
)


def _layer2_body(d0_ref, d1_ref, a0_ref, a1_ref, g1_ref, be_ref, w_ref, out_ref):
    dinv = _dinv_block(d0_ref, d1_ref)
    e = dinv * (a0_ref[...] + a1_ref[...] + g1_ref[...]) + be_ref[...]
    out_ref[...] = jnp.dot(e, w_ref[...],
                           preferred_element_type=jnp.float32) * dinv


_layer2 = pl.pallas_call(
    _layer2_body,
    grid=(GRID,),
    in_specs=[
        pl.BlockSpec((BLK, 1), lambda i: (i, 0)),
        pl.BlockSpec((BLK, 1), lambda i: (i, 0)),
        pl.BlockSpec((BLK, D), lambda i: (i, 0)),
        pl.BlockSpec((BLK, D), lambda i: (i, 0)),
        pl.BlockSpec((BLK, D), lambda i: (i, 0)),
        pl.BlockSpec((D,), lambda i: (0,)),
        pl.BlockSpec((D, D), lambda i: (0, 0)),
    ],
    out_specs=pl.BlockSpec((BLK, D), lambda i: (i, 0)),
    out_shape=jax.ShapeDtypeStruct((N, D), jnp.float32),
)


def _final_body(d0_ref, d1_ref, a0_ref, a1_ref, g2_ref, bd_ref, out_ref):
    dinv = _dinv_block(d0_ref, d1_ref)
    out_ref[...] = dinv * (a0_ref[...] + a1_ref[...] + g2_ref[...]) + bd_ref[...]


_final = pl.pallas_call(
    _final_body,
    grid=(GRID,),
    in_specs=[
        pl.BlockSpec((BLK, 1), lambda i: (i, 0)),
        pl.BlockSpec((BLK, 1), lambda i: (i, 0)),
        pl.BlockSpec((BLK, D), lambda i: (i, 0)),
        pl.BlockSpec((BLK, D), lambda i: (i, 0)),
        pl.BlockSpec((BLK, D), lambda i: (i, 0)),
        pl.BlockSpec((D,), lambda i: (0,)),
    ],
    out_specs=pl.BlockSpec((BLK, D), lambda i: (i, 0)),
    out_shape=jax.ShapeDtypeStruct((N, D), jnp.float32),
)


def kernel(x, edge_index, W_enc, b_enc, W_dec, b_dec):
    ei = edge_index.astype(jnp.int32)
    src = ei[0]
    dst = ei[1]
    ones = jnp.ones((CH,), jnp.float32)
    zeros1 = jnp.zeros((DSTRIPE,), jnp.float32)
    zeros2 = jnp.zeros((ZR, D), jnp.float32)

    deg0, deg1 = _deg_kernel(dst, ones, zeros1)
    d0 = deg0.reshape(DPAD, 1)
    d1 = deg1.reshape(DPAD, 1)
    g1 = _mm_scale(d0, d1, x, W_enc)
    a10, a11 = _spmm_kernel(g1, src, dst, zeros2)
    g2 = _layer2(d0, d1, a10, a11, g1, b_enc, W_dec)
    a20, a21 = _spmm_kernel(g2, src, dst, zeros2)
    return _final(d0, d1, a20, a21, g2, b_dec)
